# baseline (speedup 1.0000x reference)
import functools

import jax
import jax.numpy as jnp
from jax import lax
from jax.experimental import pallas as pl
from jax.experimental.pallas import tpu as pltpu
from jax.experimental.pallas import tpu_sc as plsc

B = 16384
IN = 64
H = 64
C = 64
M = 1000000

NC = 2
NS = 16
NW = NC * NS
L = 16

@functools.cache
def _mesh():
    return plsc.VectorSubcoreMesh(
        core_axis_name="c", subcore_axis_name="s", num_cores=NC,
        num_subcores=NS)


GB = B // NW
GCH = 128
GNC = GB // GCH


def _sc_gather_body(table_hbm, idx_hbm, out_hbm, idx_v, rows_v, sem):
    wid = lax.axis_index("s") * NC + lax.axis_index("c")
    pltpu.sync_copy(idx_hbm.at[pl.ds(wid * GNC, GNC)], idx_v)
    copies = [
        pltpu.async_copy(
            table_hbm.at[idx_v.at[j]], rows_v.at[pl.ds(j * GCH, GCH)], sem
        )
        for j in range(GNC)
    ]
    for cp in copies:
        cp.wait()
    pltpu.sync_copy(rows_v, out_hbm.at[pl.ds(wid * GB, GB)])


@functools.cache
def _sc_gather_kernel():
    return pl.kernel(
        _sc_gather_body,
        mesh=_mesh(),
        out_type=jax.ShapeDtypeStruct((B, H), jnp.float32),
        scratch_types=[
            pltpu.VMEM((GNC, GCH), jnp.int32),
            pltpu.VMEM((GB, H), jnp.float32),
            pltpu.SemaphoreType.DMA,
        ],
    )


RB = 1024


def _tc_gru_body(x_ref, h_ref, wir, wiz, win, whr, whz, whn, br, bz, bin_,
                 bhn, wout, bout, newh_ref, logit_ref):
    x = x_ref[...]
    h = h_ref[...]
    f32 = jnp.float32
    r = jax.nn.sigmoid(
        jnp.dot(x, wir[...], preferred_element_type=f32)
        + jnp.dot(h, whr[...], preferred_element_type=f32) + br[...])
    z = jax.nn.sigmoid(
        jnp.dot(x, wiz[...], preferred_element_type=f32)
        + jnp.dot(h, whz[...], preferred_element_type=f32) + bz[...])
    n = jnp.tanh(
        jnp.dot(x, win[...], preferred_element_type=f32) + bin_[...]
        + r * (jnp.dot(h, whn[...], preferred_element_type=f32) + bhn[...]))
    nh = (1.0 - z) * n + z * h
    newh_ref[...] = nh
    logit_ref[...] = (
        jnp.dot(nh, wout[...], preferred_element_type=f32) + bout[...])


def _tc_gru(x, h, wir, wiz, win, whr, whz, whn, br, bz, bin_, bhn, wout,
            bout):
    row = pl.BlockSpec((RB, H), lambda i: (i, 0))
    wsp = pl.BlockSpec((H, H), lambda i: (0, 0))
    bsp = pl.BlockSpec((1, H), lambda i: (0, 0))
    return pl.pallas_call(
        _tc_gru_body,
        grid=(B // RB,),
        in_specs=[row, row, wsp, wsp, wsp, wsp, wsp, wsp, bsp, bsp, bsp,
                  bsp, wsp, bsp],
        out_specs=[row, row],
        out_shape=[
            jax.ShapeDtypeStruct((B, H), jnp.float32),
            jax.ShapeDtypeStruct((B, C), jnp.float32),
        ],
    )(x, h, wir, wiz, win, whr, whz, whn, br, bz, bin_, bhn, wout, bout)


CPB = 8000


def _tc_copy_body(src_ref, dst_ref):
    dst_ref[...] = src_ref[...]


def _tc_copy(table):
    spec = pl.BlockSpec((CPB, H), lambda i: (i, 0))
    return pl.pallas_call(
        _tc_copy_body,
        grid=(M // CPB,),
        in_specs=[spec],
        out_specs=spec,
        out_shape=jax.ShapeDtypeStruct((M, H), jnp.float32),
    )(table)


RNG = M // NW
WPAD = 31264
SEL = 2064
SCH = 16


def _sc_scatter_body(idx_hbm, newh_hbm, table_ref, idx_all, winner, pos_buf,
                     row_buf, rows_v, gsem, ssem):
    wid = lax.axis_index("s") * NC + lax.axis_index("c")
    lo = wid * RNG
    iota = lax.iota(jnp.int32, L)

    pltpu.sync_copy(idx_hbm, idx_all)

    minus1 = jnp.full((L,), -1, jnp.int32)

    def init_step(t, carry):
        winner[pl.ds(t * L, L)] = minus1
        return carry

    lax.fori_loop(0, WPAD // L, init_step, 0)

    def build_step(k, carry):
        ids = idx_all[pl.ds(k * L, L)]
        m = (ids >= lo) & (ids < lo + RNG)

        @pl.when(jnp.any(m))
        def _():
            pos = iota + k * L
            rel = jnp.where(m, ids - lo, 0)

            def body(keep_going):
                cur = plsc.load_gather(winner, [rel], mask=m)
                plsc.store_scatter(winner, [rel], pos, mask=m & (cur < pos))
                chk = plsc.load_gather(winner, [rel], mask=m)
                return jnp.any(m & (chk < pos))

            lax.while_loop(lambda kg: kg, body, True)

        return carry

    lax.fori_loop(0, B // L, build_step, 0)

    def compact_step(t, cnt):
        w = winner[pl.ds(t * L, L)]
        m = w >= 0
        c = jnp.sum(m.astype(jnp.int32))

        @pl.when((c > 0) & (cnt <= SEL - L))
        def _():
            plsc.store_compressed(pos_buf.at[pl.ds(cnt, L)], w, mask=m)
            rows = iota + (lo + t * L)
            plsc.store_compressed(row_buf.at[pl.ds(cnt, L)], rows, mask=m)

        return cnt + c

    cnt = lax.fori_loop(0, WPAD // L, compact_step, 0)

    @pl.when(cnt > 0)
    def _():
        zero16 = jnp.zeros((L,), jnp.int32)
        e_r = row_buf[pl.ds(0, L)].at[zero16].get(mode="promise_in_bounds")
        e_p = pos_buf[pl.ds(0, L)].at[zero16].get(mode="promise_in_bounds")
        row_buf[pl.ds(cnt, L)] = e_r
        pos_buf[pl.ds(cnt, L)] = e_p

    nch = (cnt + SCH - 1) // SCH

    def scatter_step(c2, carry):
        pos_v = pos_buf[pl.ds(c2 * SCH, SCH)]
        row_v = row_buf[pl.ds(c2 * SCH, SCH)]
        pltpu.async_copy(newh_hbm.at[pos_v], rows_v, gsem).wait()
        pltpu.async_copy(rows_v, table_ref.at[row_v], ssem).wait()
        return carry

    lax.fori_loop(0, nch, scatter_step, 0)


@functools.cache
def _sc_scatter_kernel():
    return pl.kernel(
        _sc_scatter_body,
        mesh=_mesh(),
        out_type=(),
        scratch_types=[
            pltpu.VMEM((B,), jnp.int32),
            pltpu.VMEM((WPAD,), jnp.int32),
            pltpu.VMEM((SEL,), jnp.int32),
            pltpu.VMEM((SEL,), jnp.int32),
            pltpu.VMEM((SCH, H), jnp.float32),
            pltpu.SemaphoreType.DMA,
            pltpu.SemaphoreType.DMA,
        ],
    )


def kernel(features, node_ids, hidden_state, W_ih, W_hh, b_ih, b_hh, W_out,
           b_out):
    ids = node_ids.astype(jnp.int32)
    prev_h = jnp.take(hidden_state, ids, axis=0)

    wir, wiz, win = (W_ih[0:H].T, W_ih[H:2 * H].T, W_ih[2 * H:].T)
    whr, whz, whn = (W_hh[0:H].T, W_hh[H:2 * H].T, W_hh[2 * H:].T)
    br = (b_ih[0:H] + b_hh[0:H]).reshape(1, H)
    bz = (b_ih[H:2 * H] + b_hh[H:2 * H]).reshape(1, H)
    bin_ = b_ih[2 * H:].reshape(1, H)
    bhn = b_hh[2 * H:].reshape(1, H)

    new_h, logits = _tc_gru(features, prev_h, wir, wiz, win, whr, whz, whn,
                            br, bz, bin_, bhn, W_out.T, b_out.reshape(1, C))

    updated = hidden_state.at[ids].set(new_h)
    return logits, updated


# speedup vs baseline: 3.1752x; 3.1752x over previous
import functools

import jax
import jax.numpy as jnp
from jax import lax
from jax.experimental import pallas as pl
from jax.experimental.pallas import tpu as pltpu
from jax.experimental.pallas import tpu_sc as plsc

B = 16384
IN = 64
H = 64
C = 64
M = 1000000
WD = 128

NC = 2
NS = 16
NW = NC * NS
L = 16


@functools.cache
def _mesh():
    return plsc.VectorSubcoreMesh(
        core_axis_name="c", subcore_axis_name="s", num_cores=NC,
        num_subcores=NS)


PBN = 8192
PG = -(-M // PBN)


def _pack_body(src_ref, dst_ref):
    t = jnp.transpose(src_ref[...], (1, 0))
    dst_ref[...] = jnp.concatenate(
        [t, jnp.zeros((PBN, WD - H), jnp.float32)], axis=1)


def _tc_pack(hidT):
    return pl.pallas_call(
        _pack_body,
        grid=(PG,),
        in_specs=[pl.BlockSpec((H, PBN), lambda i: (0, i))],
        out_specs=pl.BlockSpec((PBN, WD), lambda i: (i, 0)),
        out_shape=jax.ShapeDtypeStruct((M, WD), jnp.float32),
    )(hidT)


def _unpack_body(src_ref, dst_ref):
    dst_ref[...] = jnp.transpose(src_ref[...][:, 0:H], (1, 0))


def _tc_unpack(w):
    return pl.pallas_call(
        _unpack_body,
        grid=(PG,),
        in_specs=[pl.BlockSpec((PBN, WD), lambda i: (i, 0))],
        out_specs=pl.BlockSpec((H, PBN), lambda i: (0, i)),
        out_shape=jax.ShapeDtypeStruct((H, M), jnp.float32),
    )(w)


GB = B // NW
GCH = 128
GNC = GB // GCH


def _sc_gather_body(w_hbm, idx_hbm, out_hbm, idx_v, rows_v, sem):
    wid = lax.axis_index("s") * NC + lax.axis_index("c")
    pltpu.sync_copy(idx_hbm.at[pl.ds(wid * GNC, GNC)], idx_v)
    copies = [
        pltpu.async_copy(
            w_hbm.at[idx_v.at[j]], rows_v.at[pl.ds(j * GCH, GCH)], sem
        )
        for j in range(GNC)
    ]
    for cp in copies:
        cp.wait()
    pltpu.sync_copy(rows_v, out_hbm.at[pl.ds(wid * GB, GB)])


@functools.cache
def _sc_gather_kernel():
    return pl.kernel(
        _sc_gather_body,
        mesh=_mesh(),
        out_type=jax.ShapeDtypeStruct((B, WD), jnp.float32),
        scratch_types=[
            pltpu.VMEM((GNC, GCH), jnp.int32),
            pltpu.VMEM((GB, WD), jnp.float32),
            pltpu.SemaphoreType.DMA,
        ],
        compiler_params=pltpu.CompilerParams(needs_layout_passes=False),
    )


RB = 2048


def _tc_gru_body(x_ref, hp_ref, wir, wiz, win, whr, whz, whn, br, bz, bin_,
                 bhn, wout, bout, newh_ref, logit_ref):
    x = x_ref[...]
    h = hp_ref[...][:, 0:H]
    f32 = jnp.float32
    r = jax.nn.sigmoid(
        jnp.dot(x, wir[...], preferred_element_type=f32)
        + jnp.dot(h, whr[...], preferred_element_type=f32) + br[...])
    z = jax.nn.sigmoid(
        jnp.dot(x, wiz[...], preferred_element_type=f32)
        + jnp.dot(h, whz[...], preferred_element_type=f32) + bz[...])
    n = jnp.tanh(
        jnp.dot(x, win[...], preferred_element_type=f32) + bin_[...]
        + r * (jnp.dot(h, whn[...], preferred_element_type=f32) + bhn[...]))
    nh = (1.0 - z) * n + z * h
    newh_ref[...] = jnp.concatenate(
        [nh, jnp.zeros((RB, WD - H), f32)], axis=1)
    logit_ref[...] = (
        jnp.dot(nh, wout[...], preferred_element_type=f32) + bout[...])


def _tc_gru(x, hp, wir, wiz, win, whr, whz, whn, br, bz, bin_, bhn, wout,
            bout):
    xsp = pl.BlockSpec((RB, H), lambda i: (i, 0))
    hsp = pl.BlockSpec((RB, WD), lambda i: (i, 0))
    wsp = pl.BlockSpec((H, H), lambda i: (0, 0))
    bsp = pl.BlockSpec((1, H), lambda i: (0, 0))
    return pl.pallas_call(
        _tc_gru_body,
        grid=(B // RB,),
        in_specs=[xsp, hsp, wsp, wsp, wsp, wsp, wsp, wsp, bsp, bsp, bsp,
                  bsp, wsp, bsp],
        out_specs=[hsp, xsp],
        out_shape=[
            jax.ShapeDtypeStruct((B, WD), jnp.float32),
            jax.ShapeDtypeStruct((B, C), jnp.float32),
        ],
    )(x, hp, wir, wiz, win, whr, whz, whn, br, bz, bin_, bhn, wout, bout)


RNG = M // NW
WPAD = 31264
SEL = 2064
SCH = 16


def _sc_scatter_body(idx_hbm, newh_hbm, w_ref, idx_all, winner, pos_buf,
                     row_buf, rows_v, gsem, ssem):
    wid = lax.axis_index("s") * NC + lax.axis_index("c")
    lo = wid * RNG
    iota = lax.iota(jnp.int32, L)

    pltpu.sync_copy(idx_hbm, idx_all)

    minus1 = jnp.full((L,), -1, jnp.int32)

    def init_step(t, carry):
        winner[pl.ds(t * L, L)] = minus1
        return carry

    lax.fori_loop(0, WPAD // L, init_step, 0)

    def build_step(k, carry):
        ids = idx_all[pl.ds(k * L, L)]
        m = (ids >= lo) & (ids < lo + RNG)

        @pl.when(jnp.any(m))
        def _():
            pos = iota + k * L
            rel = jnp.where(m, ids - lo, 0)

            def body(keep_going):
                cur = plsc.load_gather(winner, [rel], mask=m)
                plsc.store_scatter(winner, [rel], pos, mask=m & (cur < pos))
                chk = plsc.load_gather(winner, [rel], mask=m)
                return jnp.any(m & (chk < pos))

            lax.while_loop(lambda kg: kg, body, True)

        return carry

    lax.fori_loop(0, B // L, build_step, 0)

    def compact_step(t, cnt):
        w = winner[pl.ds(t * L, L)]
        m = w >= 0
        c = jnp.sum(m.astype(jnp.int32))

        @pl.when((c > 0) & (cnt <= SEL - L))
        def _():
            plsc.store_compressed(pos_buf.at[pl.ds(cnt, L)], w, mask=m)
            rows = iota + (lo + t * L)
            plsc.store_compressed(row_buf.at[pl.ds(cnt, L)], rows, mask=m)

        return cnt + c

    cnt = lax.fori_loop(0, WPAD // L, compact_step, 0)

    @pl.when(cnt > 0)
    def _():
        zero16 = jnp.zeros((L,), jnp.int32)
        e_r = row_buf[pl.ds(0, L)].at[zero16].get(mode="promise_in_bounds")
        e_p = pos_buf[pl.ds(0, L)].at[zero16].get(mode="promise_in_bounds")
        row_buf[pl.ds(cnt, L)] = e_r
        pos_buf[pl.ds(cnt, L)] = e_p

    nch = (cnt + SCH - 1) // SCH

    def scatter_step(c2, carry):
        pos_v = pos_buf[pl.ds(c2 * SCH, SCH)]
        row_v = row_buf[pl.ds(c2 * SCH, SCH)]
        pltpu.async_copy(newh_hbm.at[pos_v], rows_v, gsem).wait()
        pltpu.async_copy(rows_v, w_ref.at[row_v], ssem).wait()
        return carry

    lax.fori_loop(0, nch, scatter_step, 0)


@functools.cache
def _sc_scatter_kernel():
    return pl.kernel(
        _sc_scatter_body,
        mesh=_mesh(),
        out_type=(),
        scratch_types=[
            pltpu.VMEM((B,), jnp.int32),
            pltpu.VMEM((WPAD,), jnp.int32),
            pltpu.VMEM((SEL,), jnp.int32),
            pltpu.VMEM((SEL,), jnp.int32),
            pltpu.VMEM((SCH, WD), jnp.float32),
            pltpu.SemaphoreType.DMA,
            pltpu.SemaphoreType.DMA,
        ],
        compiler_params=pltpu.CompilerParams(needs_layout_passes=False),
    )


def kernel(features, node_ids, hidden_state, W_ih, W_hh, b_ih, b_hh, W_out,
           b_out):
    ids = node_ids.astype(jnp.int32)

    w_table = _tc_pack(hidden_state.T)
    prev_pad = _sc_gather_kernel()(w_table, ids.reshape(B // GCH, GCH))

    wir, wiz, win = (W_ih[0:H].T, W_ih[H:2 * H].T, W_ih[2 * H:].T)
    whr, whz, whn = (W_hh[0:H].T, W_hh[H:2 * H].T, W_hh[2 * H:].T)
    br = (b_ih[0:H] + b_hh[0:H]).reshape(1, H)
    bz = (b_ih[H:2 * H] + b_hh[H:2 * H]).reshape(1, H)
    bin_ = b_ih[2 * H:].reshape(1, H)
    bhn = b_hh[2 * H:].reshape(1, H)

    new_h_pad, logits = _tc_gru(features, prev_pad, wir, wiz, win, whr, whz,
                                whn, br, bz, bin_, bhn, W_out.T,
                                b_out.reshape(1, C))

    w_ref = jax.new_ref(w_table)
    _sc_scatter_kernel()(ids, new_h_pad, w_ref)
    return logits, _tc_unpack(w_ref[...]).T
